# baseline (speedup 1.0000x reference)
import dataclasses
import functools

import jax
import jax.numpy as jnp
from jax.experimental import pallas as pl
from jax.experimental.pallas import tpu as pltpu
from jax.experimental.pallas import tpu_sc as plsc

N = 10000
D = 256
HD = 128
NP = 10240
E_RAW = 160000
E = E_RAW + N
K = 112
NCA = 96
EPA = 16 * K * NCA
NCB = 90
EPB = 16 * K * NCB
BM = 1024
GI = NP // BM
ROWS_PER_TILE = NP // 16

_f32 = jnp.float32


_bf16 = jnp.bfloat16


def _tc1_body(x_ref, w_ref, as_ref, ad_ref, h_ref, s_ref, d_ref):
    j = pl.program_id(1)
    xb = x_ref[...]
    wj = w_ref[:, pl.ds(j * HD, HD)]
    h_ref[...] = jnp.dot(xb.astype(_bf16), wj.astype(_bf16),
                         preferred_element_type=_f32)

    @pl.when(j == 0)
    def _():
        wb = w_ref[...]
        va = jnp.dot(wb, as_ref[...], preferred_element_type=_f32)
        vd = jnp.dot(wb, ad_ref[...], preferred_element_type=_f32)
        s_ref[...] = jnp.dot(xb, va, preferred_element_type=_f32)
        d_ref[...] = jnp.dot(xb, vd, preferred_element_type=_f32)


def _tc1(x_pad, W, a_s, a_d):
    return pl.pallas_call(
        _tc1_body,
        grid=(GI, 2),
        in_specs=[
            pl.BlockSpec((BM, D), lambda i, j: (i, 0)),
            pl.BlockSpec((D, D), lambda i, j: (0, 0)),
            pl.BlockSpec((D, 1), lambda i, j: (0, 0)),
            pl.BlockSpec((D, 1), lambda i, j: (0, 0)),
        ],
        out_specs=[
            pl.BlockSpec((BM, HD), lambda i, j: (i + j * GI, 0)),
            pl.BlockSpec((BM, 1), lambda i, j: (i, 0)),
            pl.BlockSpec((BM, 1), lambda i, j: (i, 0)),
        ],
        out_shape=[
            jax.ShapeDtypeStruct((2 * NP, HD), _f32),
            jax.ShapeDtypeStruct((NP, 1), _f32),
            jax.ShapeDtypeStruct((NP, 1), _f32),
        ],
    )(x_pad, W, a_s.reshape(D, 1), a_d.reshape(D, 1))


def _make_tc_mid_body(pre_scale):
    def body(lo_ref, hi_ref, dn_ref, dg_ref, b_ref, w_ref, out_ref):
        j = pl.program_id(1)
        lo = lo_ref[...]
        hi = hi_ref[...]
        dg = dg_ref[...]
        dinv = jnp.where(dg > 0, jax.lax.rsqrt(dg), 0.0)
        if pre_scale:
            a_lo = jax.nn.relu(dinv * lo + b_ref[0:1, :HD])
            a_hi = jax.nn.relu(dinv * hi + b_ref[0:1, HD:])
        else:
            dn = dn_ref[...] + 1e-16
            a_lo = jax.nn.relu(lo / dn + b_ref[0:1, :HD])
            a_hi = jax.nn.relu(hi / dn + b_ref[0:1, HD:])
        wlo = w_ref[0:HD, pl.ds(j * HD, HD)].astype(_bf16)
        whi = w_ref[HD:D, pl.ds(j * HD, HD)].astype(_bf16)
        out_ref[...] = dinv * (
            jnp.dot(a_lo.astype(_bf16), wlo, preferred_element_type=_f32)
            + jnp.dot(a_hi.astype(_bf16), whi, preferred_element_type=_f32))
    return body


def _tc_mid(acc, dn, dg, b, W, pre_scale):
    return pl.pallas_call(
        _make_tc_mid_body(pre_scale),
        grid=(GI, 2),
        in_specs=[
            pl.BlockSpec((BM, HD), lambda i, j: (i, 0)),
            pl.BlockSpec((BM, HD), lambda i, j: (i + GI, 0)),
            pl.BlockSpec((BM, 1), lambda i, j: (i, 0)),
            pl.BlockSpec((BM, 1), lambda i, j: (i, 0)),
            pl.BlockSpec((1, D), lambda i, j: (0, 0)),
            pl.BlockSpec((D, D), lambda i, j: (0, 0)),
        ],
        out_specs=pl.BlockSpec((BM, HD), lambda i, j: (i + j * GI, 0)),
        out_shape=jax.ShapeDtypeStruct((2 * NP, HD), _f32),
    )(acc, acc, dn, dg, b.reshape(1, D), W)


def _tc4_body(lo_ref, hi_ref, dg_ref, b_ref, out_ref):
    dg = dg_ref[...]
    dinv = jnp.where(dg > 0, jax.lax.rsqrt(dg), 0.0)
    h = jnp.concatenate([lo_ref[...], hi_ref[...]], axis=1)
    out_ref[...] = dinv * h + b_ref[...]


def _tc4(acc, dg, b):
    return pl.pallas_call(
        _tc4_body,
        grid=(GI,),
        in_specs=[
            pl.BlockSpec((BM, HD), lambda i: (i, 0)),
            pl.BlockSpec((BM, HD), lambda i: (i + GI, 0)),
            pl.BlockSpec((BM, 1), lambda i: (i, 0)),
            pl.BlockSpec((1, D), lambda i: (0, 0)),
        ],
        out_specs=pl.BlockSpec((BM, D), lambda i: (i, 0)),
        out_shape=jax.ShapeDtypeStruct((N, D), _f32),
    )(acc, acc, dg, b.reshape(1, D))


_MESH = plsc.VectorSubcoreMesh(core_axis_name="c", subcore_axis_name="s")

_SC_PARAMS = pltpu.CompilerParams()
if "needs_layout_passes" in pltpu.CompilerParams.__dataclass_fields__:
    _SC_PARAMS = dataclasses.replace(_SC_PARAMS, needs_layout_passes=False)


def _sc_gat(src2, dst2, as2v, adv, h_split, z_rows, z_vec):
    R = 3
    scratch = [
        pltpu.VMEM_SHARED((NP, HD), _f32),
        pltpu.VMEM_SHARED((NP,), _f32),
        pltpu.VMEM_SHARED((NP,), _f32),
        pltpu.VMEM((K,), _f32),
    ]
    for _ in range(R):
        scratch += [
            pltpu.VMEM((K,), jnp.int32),
            pltpu.VMEM((K,), jnp.int32),
            pltpu.VMEM((K,), _f32),
            pltpu.VMEM((K,), _f32),
            pltpu.VMEM((K,), _f32),
            pltpu.VMEM((K, HD), _f32),
            pltpu.SemaphoreType.DMA,
            pltpu.SemaphoreType.DMA,
            pltpu.SemaphoreType.DMA,
            pltpu.VMEM((K,), jnp.int32),
        ]

    @functools.partial(
        pl.kernel,
        out_type=[
            jax.ShapeDtypeStruct((2 * NP, HD), _f32),
            jax.ShapeDtypeStruct((NP,), _f32),
            jax.ShapeDtypeStruct((NP,), _f32),
        ],
        mesh=_MESH,
        scratch_types=scratch,
        compiler_params=_SC_PARAMS,
    )
    def k(src_hbm, dst_hbm, as_hbm, ad_hbm, h_hbm, zr_hbm, zv_hbm,
          num_hbm, den_hbm, deg_hbm, acc_sp, den_sp, deg_sp, onesv, *rest):
        bufs = [rest[10 * b:10 * b + 10] for b in range(R)]
        c = jax.lax.axis_index("c")
        s = jax.lax.axis_index("s")
        pltpu.sync_copy(zr_hbm, acc_sp.at[pl.ds(s * ROWS_PER_TILE, ROWS_PER_TILE)])
        pltpu.sync_copy(zv_hbm, den_sp.at[pl.ds(s * ROWS_PER_TILE, ROWS_PER_TILE)])
        pltpu.sync_copy(zv_hbm, deg_sp.at[pl.ds(s * ROWS_PER_TILE, ROWS_PER_TILE)])
        for g in range(K // 16):
            onesv[pl.ds(g * 16, 16)] = jnp.full((16,), 1.0, _f32)
        plsc.subcore_barrier()

        def start_idx(j, b):
            srcb, dstb = bufs[b][0], bufs[b][1]
            isem = bufs[b][6]
            pltpu.async_copy(src_hbm.at[s, j], srcb, isem)
            pltpu.async_copy(dst_hbm.at[s, j], dstb, isem)

        def wait_idx(b):
            srcb, dstb = bufs[b][0], bufs[b][1]
            isem = bufs[b][6]
            pltpu.make_async_copy(src_hbm.at[s, 0], srcb, isem).wait()
            pltpu.make_async_copy(dst_hbm.at[s, 0], dstb, isem).wait()

        def start_gather(b):
            srcb, dstb, asvb, advb, _, rowb, _, gsem, _, srco = bufs[b]
            for g in range(K // 16):
                sl = pl.ds(g * 16, 16)
                srco[sl] = srcb[sl] + c * NP
            pltpu.async_copy(h_hbm.at[srco], rowb, gsem)
            pltpu.async_copy(as_hbm.at[srcb], asvb, gsem)
            pltpu.async_copy(ad_hbm.at[dstb], advb, gsem)

        def wait_gather(b):
            srcb, dstb, asvb, advb, _, rowb, _, gsem, _, srco = bufs[b]
            pltpu.make_async_copy(h_hbm.at[srco], rowb, gsem).wait()
            pltpu.make_async_copy(as_hbm.at[srcb], asvb, gsem).wait()
            pltpu.make_async_copy(ad_hbm.at[dstb], advb, gsem).wait()

        def wait_dsem(b):
            _, dstb, _, _, wv, _, _, _, dsem, _ = bufs[b]
            pltpu.make_async_copy(wv, den_sp.at[dstb], dsem).wait()
            pltpu.make_async_copy(onesv, deg_sp.at[dstb], dsem).wait()

        for b in range(R):
            start_idx(b, b)
        wait_idx(0)
        start_gather(0)
        wait_idx(1)
        start_gather(1)

        @pl.loop(0, NCA, step=R)
        def _(jo):
            for b in range(R):
                j = jo + b
                srcb, dstb, asvb, advb, wv, rowb, isem, gsem, dsem, srco = bufs[b]
                wait_gather(b)

                @pl.when(j >= R)
                def _():
                    wait_dsem(b)

                @pl.loop(0, K // 16)
                def _(g):
                    sl = pl.ds(g * 16, 16)
                    e = asvb[sl] + advb[sl]
                    e = jnp.where(e < 0.0, e * 0.2, e)
                    wv[sl] = jnp.exp(e)

                @pl.loop(0, K, unroll=4)
                def _(eid):
                    wq = plsc.load_gather(wv, [jnp.full((16,), eid, jnp.int32)])
                    for q in range(HD // 16):
                        rowb[eid, pl.ds(q * 16, 16)] = (
                            rowb[eid, pl.ds(q * 16, 16)] * wq)

                nb = (b + 2) % R

                @pl.when(j + 2 < NCA)
                def _():
                    wait_idx(nb)
                    start_gather(nb)

                pltpu.async_copy(wv, den_sp.at[dstb], dsem, add=True)
                pltpu.async_copy(onesv, deg_sp.at[dstb], dsem, add=True)
                pltpu.sync_copy(rowb, acc_sp.at[dstb], add=True)

                @pl.when(j + R < NCA)
                def _():
                    start_idx(j + R, b)

        for b in range(R):
            wait_dsem(b)
        plsc.subcore_barrier()
        rs = pl.ds(s * ROWS_PER_TILE, ROWS_PER_TILE)
        pltpu.sync_copy(acc_sp.at[rs],
                        num_hbm.at[pl.ds(c * NP + s * ROWS_PER_TILE, ROWS_PER_TILE)])

        @pl.when(c == 0)
        def _():
            pltpu.sync_copy(den_sp.at[rs], den_hbm.at[rs])
            pltpu.sync_copy(deg_sp.at[rs], deg_hbm.at[rs])

    return k(src2, dst2, as2v, adv, h_split, z_rows, z_vec)


def _sc_gcn(src2, dst2, h_split):
    R = 3
    scratch = [pltpu.VMEM_SHARED((NP, HD), _f32)]
    for _ in range(R):
        scratch += [
            pltpu.VMEM((K,), jnp.int32),
            pltpu.VMEM((K,), jnp.int32),
            pltpu.VMEM((K, HD), _f32),
            pltpu.SemaphoreType.DMA,
            pltpu.SemaphoreType.DMA,
        ]

    @functools.partial(
        pl.kernel,
        out_type=jax.ShapeDtypeStruct((2 * NP, HD), _f32),
        mesh=_MESH,
        scratch_types=scratch,
        compiler_params=_SC_PARAMS,
    )
    def k(src_hbm, dst_hbm, h_hbm, out_hbm, acc_sp, *rest):
        bufs = [rest[5 * b:5 * b + 5] for b in range(R)]
        c = jax.lax.axis_index("c")
        s = jax.lax.axis_index("s")
        rs = pl.ds(s * ROWS_PER_TILE, ROWS_PER_TILE)
        pltpu.sync_copy(
            h_hbm.at[pl.ds(c * NP + s * ROWS_PER_TILE, ROWS_PER_TILE)],
            acc_sp.at[rs])
        plsc.subcore_barrier()

        def start_idx(j, b):
            srcb, dstb, _, isem, _ = bufs[b]
            pltpu.async_copy(src_hbm.at[s, j], srcb, isem)
            pltpu.async_copy(dst_hbm.at[s, j], dstb, isem)

        def wait_idx(b):
            srcb, dstb, _, isem, _ = bufs[b]
            pltpu.make_async_copy(src_hbm.at[s, 0], srcb, isem).wait()
            pltpu.make_async_copy(dst_hbm.at[s, 0], dstb, isem).wait()

        def start_gather(b):
            srcb, _, rowb, _, gsem = bufs[b]
            for g in range(K // 16):
                sl = pl.ds(g * 16, 16)
                srcb[sl] = srcb[sl] + c * NP
            pltpu.async_copy(h_hbm.at[srcb], rowb, gsem)

        def wait_gather(b):
            srcb, _, rowb, _, gsem = bufs[b]
            pltpu.make_async_copy(h_hbm.at[srcb], rowb, gsem).wait()

        for b in range(R):
            start_idx(b, b)
        wait_idx(0)
        start_gather(0)
        wait_idx(1)
        start_gather(1)

        @pl.loop(0, NCB, step=R)
        def _(jo):
            for b in range(R):
                j = jo + b
                srcb, dstb, rowb, isem, gsem = bufs[b]
                wait_gather(b)
                nb = (b + 2) % R

                @pl.when(j + 2 < NCB)
                def _():
                    wait_idx(nb)
                    start_gather(nb)

                pltpu.sync_copy(rowb, acc_sp.at[dstb], add=True)

                @pl.when(j + R < NCB)
                def _():
                    start_idx(j + R, b)

        plsc.subcore_barrier()
        pltpu.sync_copy(acc_sp.at[rs],
                        out_hbm.at[pl.ds(c * NP + s * ROWS_PER_TILE, ROWS_PER_TILE)])

    return k(src2, dst2, h_split)


def kernel(x, edge_index, W_gat, att_src, att_dst, b_gat, W1, b1, W2, b2):
    ei = edge_index.astype(jnp.int32)
    loops = jnp.arange(N, dtype=jnp.int32)
    pad_a = EPA - E
    src2a = jnp.concatenate(
        [ei[0], loops, jnp.zeros((pad_a,), jnp.int32)]).reshape(16, NCA, K)
    dst2a = jnp.concatenate(
        [ei[1], loops, jnp.full((pad_a,), N, jnp.int32)]).reshape(16, NCA, K)
    pad_b = EPB - E_RAW
    src2b = jnp.concatenate(
        [ei[0], jnp.zeros((pad_b,), jnp.int32)]).reshape(16, NCB, K)
    dst2b = jnp.concatenate(
        [ei[1], jnp.full((pad_b,), N, jnp.int32)]).reshape(16, NCB, K)
    x_pad = jnp.pad(x, ((0, NP - N), (0, 0)))
    z_rows = jnp.zeros((ROWS_PER_TILE, HD), _f32)
    z_vec = jnp.zeros((ROWS_PER_TILE,), _f32)

    h_split, asv, adv = _tc1(x_pad, W_gat, att_src, att_dst)
    num, den, deg = _sc_gat(src2a, dst2a, asv.reshape(NP), adv.reshape(NP),
                            h_split, z_rows, z_vec)
    den2 = den.reshape(NP, 1)
    deg2 = deg.reshape(NP, 1)
    g1 = _tc_mid(num, den2, deg2, b_gat, W1, pre_scale=False)
    acc1 = _sc_gcn(src2b, dst2b, g1)
    g2 = _tc_mid(acc1, den2, deg2, b1, W2, pre_scale=True)
    acc2 = _sc_gcn(src2b, dst2b, g2)
    return _tc4(acc2, deg2, b2)


# speedup vs baseline: 1.0356x; 1.0356x over previous
import dataclasses
import functools

import jax
import jax.numpy as jnp
from jax.experimental import pallas as pl
from jax.experimental.pallas import tpu as pltpu
from jax.experimental.pallas import tpu_sc as plsc

N = 10000
D = 256
HD = 128
NP = 10240
E_RAW = 160000
E = E_RAW + N
K = 112
NCA = 96
EPA = 16 * K * NCA
NCB = 90
EPB = 16 * K * NCB
BM = 1024
GI = NP // BM
ROWS_PER_TILE = NP // 16

_f32 = jnp.float32


_bf16 = jnp.bfloat16


def _tc1_body(x_ref, wb_ref, w_ref, as_ref, ad_ref, h_ref, s_ref, d_ref):
    j = pl.program_id(1)
    xb = x_ref[...]
    wj = wb_ref[:, pl.ds(j * HD, HD)]
    h_ref[...] = jnp.dot(xb.astype(_bf16), wj, preferred_element_type=_f32)

    @pl.when(j == 0)
    def _():
        wb = w_ref[...]
        va = jnp.dot(wb, as_ref[...], preferred_element_type=_f32)
        vd = jnp.dot(wb, ad_ref[...], preferred_element_type=_f32)
        s_ref[...] = jnp.sum(xb * va.reshape(1, D), axis=1)
        d_ref[...] = jnp.sum(xb * vd.reshape(1, D), axis=1)


def _tc1(x_pad, W, Wb, a_s, a_d):
    return pl.pallas_call(
        _tc1_body,
        grid=(GI, 2),
        in_specs=[
            pl.BlockSpec((BM, D), lambda i, j: (i, 0)),
            pl.BlockSpec((D, D), lambda i, j: (0, 0)),
            pl.BlockSpec((D, D), lambda i, j: (0, 0)),
            pl.BlockSpec((D, 1), lambda i, j: (0, 0)),
            pl.BlockSpec((D, 1), lambda i, j: (0, 0)),
        ],
        out_specs=[
            pl.BlockSpec((BM, HD), lambda i, j: (i + j * GI, 0)),
            pl.BlockSpec((BM,), lambda i, j: (i,)),
            pl.BlockSpec((BM,), lambda i, j: (i,)),
        ],
        out_shape=[
            jax.ShapeDtypeStruct((2 * NP, HD), _f32),
            jax.ShapeDtypeStruct((NP,), _f32),
            jax.ShapeDtypeStruct((NP,), _f32),
        ],
    )(x_pad, Wb, W, a_s.reshape(D, 1), a_d.reshape(D, 1))


def _make_tc_mid_body(pre_scale):
    def body(lo_ref, hi_ref, dn_ref, dg_ref, b_ref, w_ref, out_ref):
        j = pl.program_id(1)
        lo = lo_ref[...]
        hi = hi_ref[...]
        dg = dg_ref[...]
        dinv = jnp.where(dg > 0, jax.lax.rsqrt(dg), 0.0)
        if pre_scale:
            a_lo = jax.nn.relu(dinv * lo + b_ref[0:1, :HD])
            a_hi = jax.nn.relu(dinv * hi + b_ref[0:1, HD:])
        else:
            dn = dn_ref[...] + 1e-16
            a_lo = jax.nn.relu(lo / dn + b_ref[0:1, :HD])
            a_hi = jax.nn.relu(hi / dn + b_ref[0:1, HD:])
        wlo = w_ref[0:HD, pl.ds(j * HD, HD)]
        whi = w_ref[HD:D, pl.ds(j * HD, HD)]
        out_ref[...] = dinv * (
            jnp.dot(a_lo.astype(_bf16), wlo, preferred_element_type=_f32)
            + jnp.dot(a_hi.astype(_bf16), whi, preferred_element_type=_f32))
    return body


def _tc_mid(acc, dn, dg, b, W, pre_scale):
    return pl.pallas_call(
        _make_tc_mid_body(pre_scale),
        grid=(GI, 2),
        in_specs=[
            pl.BlockSpec((BM, HD), lambda i, j: (i, 0)),
            pl.BlockSpec((BM, HD), lambda i, j: (i + GI, 0)),
            pl.BlockSpec((BM, 1), lambda i, j: (i, 0)),
            pl.BlockSpec((BM, 1), lambda i, j: (i, 0)),
            pl.BlockSpec((1, D), lambda i, j: (0, 0)),
            pl.BlockSpec((D, D), lambda i, j: (0, 0)),
        ],
        out_specs=pl.BlockSpec((BM, HD), lambda i, j: (i + j * GI, 0)),
        out_shape=jax.ShapeDtypeStruct((2 * NP, HD), _f32),
    )(acc, acc, dn, dg, b.reshape(1, D), W)


def _tc4_body(lo_ref, hi_ref, dg_ref, b_ref, out_ref):
    dg = dg_ref[...]
    dinv = jnp.where(dg > 0, jax.lax.rsqrt(dg), 0.0)
    h = jnp.concatenate([lo_ref[...], hi_ref[...]], axis=1)
    out_ref[...] = dinv * h + b_ref[...]


def _tc4(acc, dg, b):
    return pl.pallas_call(
        _tc4_body,
        grid=(GI,),
        in_specs=[
            pl.BlockSpec((BM, HD), lambda i: (i, 0)),
            pl.BlockSpec((BM, HD), lambda i: (i + GI, 0)),
            pl.BlockSpec((BM, 1), lambda i: (i, 0)),
            pl.BlockSpec((1, D), lambda i: (0, 0)),
        ],
        out_specs=pl.BlockSpec((BM, D), lambda i: (i, 0)),
        out_shape=jax.ShapeDtypeStruct((N, D), _f32),
    )(acc, acc, dg, b.reshape(1, D))


_MESH = plsc.VectorSubcoreMesh(core_axis_name="c", subcore_axis_name="s")

_SC_PARAMS = pltpu.CompilerParams()
if "needs_layout_passes" in pltpu.CompilerParams.__dataclass_fields__:
    _SC_PARAMS = dataclasses.replace(_SC_PARAMS, needs_layout_passes=False)


def _sc_gat(src2, dst2, as2v, adv, h_split, z_rows, z_vec):
    R = 3
    scratch = [
        pltpu.VMEM_SHARED((NP, HD), _f32),
        pltpu.VMEM_SHARED((NP,), _f32),
        pltpu.VMEM_SHARED((NP,), _f32),
        pltpu.VMEM((K,), _f32),
    ]
    for _ in range(R):
        scratch += [
            pltpu.VMEM((K,), jnp.int32),
            pltpu.VMEM((K,), jnp.int32),
            pltpu.VMEM((K,), _f32),
            pltpu.VMEM((K,), _f32),
            pltpu.VMEM((K,), _f32),
            pltpu.VMEM((K, HD), _f32),
            pltpu.SemaphoreType.DMA,
            pltpu.SemaphoreType.DMA,
            pltpu.SemaphoreType.DMA,
            pltpu.VMEM((K,), jnp.int32),
        ]

    @functools.partial(
        pl.kernel,
        out_type=[
            jax.ShapeDtypeStruct((2 * NP, HD), _f32),
            jax.ShapeDtypeStruct((NP,), _f32),
            jax.ShapeDtypeStruct((NP,), _f32),
        ],
        mesh=_MESH,
        scratch_types=scratch,
        compiler_params=_SC_PARAMS,
    )
    def k(src_hbm, dst_hbm, as_hbm, ad_hbm, h_hbm, zr_hbm, zv_hbm,
          num_hbm, den_hbm, deg_hbm, acc_sp, den_sp, deg_sp, onesv, *rest):
        bufs = [rest[10 * b:10 * b + 10] for b in range(R)]
        c = jax.lax.axis_index("c")
        s = jax.lax.axis_index("s")
        pltpu.sync_copy(zr_hbm, acc_sp.at[pl.ds(s * ROWS_PER_TILE, ROWS_PER_TILE)])
        pltpu.sync_copy(zv_hbm, den_sp.at[pl.ds(s * ROWS_PER_TILE, ROWS_PER_TILE)])
        pltpu.sync_copy(zv_hbm, deg_sp.at[pl.ds(s * ROWS_PER_TILE, ROWS_PER_TILE)])
        for g in range(K // 16):
            onesv[pl.ds(g * 16, 16)] = jnp.full((16,), 1.0, _f32)
        plsc.subcore_barrier()

        def start_idx(j, b):
            srcb, dstb = bufs[b][0], bufs[b][1]
            isem = bufs[b][6]
            pltpu.async_copy(src_hbm.at[s, j], srcb, isem)
            pltpu.async_copy(dst_hbm.at[s, j], dstb, isem)

        def wait_idx(b):
            srcb, dstb = bufs[b][0], bufs[b][1]
            isem = bufs[b][6]
            pltpu.make_async_copy(src_hbm.at[s, 0], srcb, isem).wait()
            pltpu.make_async_copy(dst_hbm.at[s, 0], dstb, isem).wait()

        def start_gather(b):
            srcb, dstb, asvb, advb, _, rowb, _, gsem, _, srco = bufs[b]
            for g in range(K // 16):
                sl = pl.ds(g * 16, 16)
                srco[sl] = srcb[sl] + c * NP
            pltpu.async_copy(h_hbm.at[srco], rowb, gsem)
            pltpu.async_copy(as_hbm.at[srcb], asvb, gsem)
            pltpu.async_copy(ad_hbm.at[dstb], advb, gsem)

        def wait_gather(b):
            srcb, dstb, asvb, advb, _, rowb, _, gsem, _, srco = bufs[b]
            pltpu.make_async_copy(h_hbm.at[srco], rowb, gsem).wait()
            pltpu.make_async_copy(as_hbm.at[srcb], asvb, gsem).wait()
            pltpu.make_async_copy(ad_hbm.at[dstb], advb, gsem).wait()

        def wait_dsem(b):
            _, dstb, _, _, wv, _, _, _, dsem, _ = bufs[b]
            pltpu.make_async_copy(wv, den_sp.at[dstb], dsem).wait()
            pltpu.make_async_copy(onesv, deg_sp.at[dstb], dsem).wait()

        for b in range(R):
            start_idx(b, b)
        wait_idx(0)
        start_gather(0)
        wait_idx(1)
        start_gather(1)

        @pl.loop(0, NCA, step=R)
        def _(jo):
            for b in range(R):
                j = jo + b
                srcb, dstb, asvb, advb, wv, rowb, isem, gsem, dsem, srco = bufs[b]
                wait_gather(b)

                @pl.when(j >= R)
                def _():
                    wait_dsem(b)

                @pl.loop(0, K // 16)
                def _(g):
                    sl = pl.ds(g * 16, 16)
                    e = asvb[sl] + advb[sl]
                    e = jnp.where(e < 0.0, e * 0.2, e)
                    wv[sl] = jnp.exp(e)

                @pl.loop(0, K)
                def _(eid):
                    wq = plsc.load_gather(wv, [jnp.full((16,), eid, jnp.int32)])
                    for q in range(HD // 16):
                        rowb[eid, pl.ds(q * 16, 16)] = (
                            rowb[eid, pl.ds(q * 16, 16)] * wq)

                nb = (b + 2) % R

                @pl.when(j + 2 < NCA)
                def _():
                    wait_idx(nb)
                    start_gather(nb)

                pltpu.async_copy(wv, den_sp.at[dstb], dsem, add=True)
                pltpu.async_copy(onesv, deg_sp.at[dstb], dsem, add=True)
                pltpu.sync_copy(rowb, acc_sp.at[dstb], add=True)

                @pl.when(j + R < NCA)
                def _():
                    start_idx(j + R, b)

        for b in range(R):
            wait_dsem(b)
        plsc.subcore_barrier()
        rs = pl.ds(s * ROWS_PER_TILE, ROWS_PER_TILE)
        pltpu.sync_copy(acc_sp.at[rs],
                        num_hbm.at[pl.ds(c * NP + s * ROWS_PER_TILE, ROWS_PER_TILE)])

        @pl.when(c == 0)
        def _():
            pltpu.sync_copy(den_sp.at[rs], den_hbm.at[rs])
            pltpu.sync_copy(deg_sp.at[rs], deg_hbm.at[rs])

    return k(src2, dst2, as2v, adv, h_split, z_rows, z_vec)


def _sc_gcn(src2, dst2, h_split):
    R = 3
    scratch = [pltpu.VMEM_SHARED((NP, HD), _f32)]
    for _ in range(R):
        scratch += [
            pltpu.VMEM((K,), jnp.int32),
            pltpu.VMEM((K,), jnp.int32),
            pltpu.VMEM((K, HD), _f32),
            pltpu.SemaphoreType.DMA,
            pltpu.SemaphoreType.DMA,
        ]

    @functools.partial(
        pl.kernel,
        out_type=jax.ShapeDtypeStruct((2 * NP, HD), _f32),
        mesh=_MESH,
        scratch_types=scratch,
        compiler_params=_SC_PARAMS,
    )
    def k(src_hbm, dst_hbm, h_hbm, out_hbm, acc_sp, *rest):
        bufs = [rest[5 * b:5 * b + 5] for b in range(R)]
        c = jax.lax.axis_index("c")
        s = jax.lax.axis_index("s")
        rs = pl.ds(s * ROWS_PER_TILE, ROWS_PER_TILE)
        pltpu.sync_copy(
            h_hbm.at[pl.ds(c * NP + s * ROWS_PER_TILE, ROWS_PER_TILE)],
            acc_sp.at[rs])
        plsc.subcore_barrier()

        def start_idx(j, b):
            srcb, dstb, _, isem, _ = bufs[b]
            pltpu.async_copy(src_hbm.at[s, j], srcb, isem)
            pltpu.async_copy(dst_hbm.at[s, j], dstb, isem)

        def wait_idx(b):
            srcb, dstb, _, isem, _ = bufs[b]
            pltpu.make_async_copy(src_hbm.at[s, 0], srcb, isem).wait()
            pltpu.make_async_copy(dst_hbm.at[s, 0], dstb, isem).wait()

        def start_gather(b):
            srcb, _, rowb, _, gsem = bufs[b]
            for g in range(K // 16):
                sl = pl.ds(g * 16, 16)
                srcb[sl] = srcb[sl] + c * NP
            pltpu.async_copy(h_hbm.at[srcb], rowb, gsem)

        def wait_gather(b):
            srcb, _, rowb, _, gsem = bufs[b]
            pltpu.make_async_copy(h_hbm.at[srcb], rowb, gsem).wait()

        for b in range(R):
            start_idx(b, b)
        wait_idx(0)
        start_gather(0)
        wait_idx(1)
        start_gather(1)

        @pl.loop(0, NCB, step=R)
        def _(jo):
            for b in range(R):
                j = jo + b
                srcb, dstb, rowb, isem, gsem = bufs[b]
                wait_gather(b)
                nb = (b + 2) % R

                @pl.when(j + 2 < NCB)
                def _():
                    wait_idx(nb)
                    start_gather(nb)

                pltpu.sync_copy(rowb, acc_sp.at[dstb], add=True)

                @pl.when(j + R < NCB)
                def _():
                    start_idx(j + R, b)

        plsc.subcore_barrier()
        pltpu.sync_copy(acc_sp.at[rs],
                        out_hbm.at[pl.ds(c * NP + s * ROWS_PER_TILE, ROWS_PER_TILE)])

    return k(src2, dst2, h_split)


def kernel(x, edge_index, W_gat, att_src, att_dst, b_gat, W1, b1, W2, b2):
    ei = edge_index.astype(jnp.int32)
    loops = jnp.arange(N, dtype=jnp.int32)
    pad_a = EPA - E
    src2a = jnp.concatenate(
        [ei[0], loops, jnp.zeros((pad_a,), jnp.int32)]).reshape(16, NCA, K)
    dst2a = jnp.concatenate(
        [ei[1], loops, jnp.full((pad_a,), N, jnp.int32)]).reshape(16, NCA, K)
    pad_b = EPB - E_RAW
    src2b = jnp.concatenate(
        [ei[0], jnp.zeros((pad_b,), jnp.int32)]).reshape(16, NCB, K)
    dst2b = jnp.concatenate(
        [ei[1], jnp.full((pad_b,), N, jnp.int32)]).reshape(16, NCB, K)
    x_pad = jnp.pad(x, ((0, NP - N), (0, 0)))
    z_rows = jnp.zeros((ROWS_PER_TILE, HD), _f32)
    z_vec = jnp.zeros((ROWS_PER_TILE,), _f32)

    h_split, asv, adv = _tc1(x_pad, W_gat, W_gat.astype(_bf16),
                             att_src, att_dst)
    num, den, deg = _sc_gat(src2a, dst2a, asv, adv, h_split, z_rows, z_vec)
    den2 = den.reshape(NP, 1)
    deg2 = deg.reshape(NP, 1)
    g1 = _tc_mid(num, den2, deg2, b_gat, W1.astype(_bf16), pre_scale=False)
    acc1 = _sc_gcn(src2b, dst2b, g1)
    g2 = _tc_mid(acc1, den2, deg2, b1, W2.astype(_bf16), pre_scale=True)
    acc2 = _sc_gcn(src2b, dst2b, g2)
    return _tc4(acc2, deg2, b2)
